# baseline (speedup 1.0000x reference)
import functools

import numpy as np
import jax
import jax.numpy as jnp
from jax import lax
from jax.experimental import pallas as pl
from jax.experimental.pallas import tpu as pltpu
from jax.experimental.pallas import tpu_sc as plsc

NL = 10000
NC = 10000
E = 160000
D = 256
NQ = 4
H = D // NQ
ITERS = 3

NCORES = 2
NPASS = NQ // NCORES
NTILES = 16
CHUNK = 128
EP_TILE = 10240
NCHUNK = EP_TILE // CHUNK
E_PAD = EP_TILE * NTILES
NBUF = 4

NSUB = 3
N_PAD = 10176
TRPT = N_PAD // NTILES
DST_H = N_PAD // NSUB
GARB = 64
ACC_ROWS = DST_H + GARB
RPT = ACC_ROWS // NTILES
WRT = DST_H // 8
EPT = E // NTILES
NITER = EPT // 16


def _part_body(ing_hbm, ink_hbm, pg_hbm, ps_hbm, bnd_hbm,
               gbuf, kbuf, og0, os0, og1, os1, og2, os2, bnd_v):
    d = lax.axis_index("c")
    t = lax.axis_index("s")
    pltpu.sync_copy(ing_hbm.at[d, t], gbuf)
    pltpu.sync_copy(ink_hbm.at[d, t], kbuf)

    zero16 = jnp.zeros((16,), jnp.int32)
    garb16 = DST_H + lax.iota(jnp.int32, 16)

    def fill(i, carry):
        og0[pl.ds(i * 16, 16)] = zero16
        og1[pl.ds(i * 16, 16)] = zero16
        og2[pl.ds(i * 16, 16)] = zero16
        os0[pl.ds(i * 16, 16)] = garb16
        os1[pl.ds(i * 16, 16)] = garb16
        os2[pl.ds(i * 16, 16)] = garb16
        return carry

    lax.fori_loop(0, EP_TILE // 16, fill, 0)

    def body(i, carry):
        o0, o1, o2 = carry
        kv = kbuf[pl.ds(i * 16, 16)]
        gv = gbuf[pl.ds(i * 16, 16)]
        m0 = kv < DST_H
        m2 = kv >= 2 * DST_H
        m1 = jnp.logical_not(jnp.logical_or(m0, m2))
        plsc.store_compressed(og0.at[pl.ds(o0, 16)], gv, mask=m0)
        plsc.store_compressed(os0.at[pl.ds(o0, 16)], kv, mask=m0)
        plsc.store_compressed(og1.at[pl.ds(o1, 16)], gv, mask=m1)
        plsc.store_compressed(os1.at[pl.ds(o1, 16)], kv - DST_H, mask=m1)
        plsc.store_compressed(og2.at[pl.ds(o2, 16)], gv, mask=m2)
        plsc.store_compressed(os2.at[pl.ds(o2, 16)], kv - 2 * DST_H, mask=m2)
        c0 = jnp.sum(m0.astype(jnp.int32))
        c2 = jnp.sum(m2.astype(jnp.int32))
        return (o0 + c0, o1 + (16 - c0 - c2), o2 + c2)

    cnt0, cnt1, cnt2 = lax.fori_loop(
        0, NITER, body, (jnp.int32(0), jnp.int32(0), jnp.int32(0)))
    bnd_v[0] = jnp.full((16,), (cnt0 + CHUNK - 1) // CHUNK, jnp.int32)
    bnd_v[1] = jnp.full((16,), (cnt1 + CHUNK - 1) // CHUNK, jnp.int32)
    bnd_v[2] = jnp.full((16,), (cnt2 + CHUNK - 1) // CHUNK, jnp.int32)

    pltpu.sync_copy(og0, pg_hbm.at[d, 0, t])
    pltpu.sync_copy(og1, pg_hbm.at[d, 1, t])
    pltpu.sync_copy(og2, pg_hbm.at[d, 2, t])
    pltpu.sync_copy(os0, ps_hbm.at[d, 0, t])
    pltpu.sync_copy(os1, ps_hbm.at[d, 1, t])
    pltpu.sync_copy(os2, ps_hbm.at[d, 2, t])
    pltpu.sync_copy(bnd_v, bnd_hbm.at[d, t])


@functools.cache
def _make_part():
    return pl.kernel(
        _part_body,
        out_type=(
            jax.ShapeDtypeStruct((2, NSUB, NTILES, EP_TILE), jnp.int32),
            jax.ShapeDtypeStruct((2, NSUB, NTILES, EP_TILE), jnp.int32),
            jax.ShapeDtypeStruct((2, NTILES, NSUB, 16), jnp.int32),
        ),
        mesh=plsc.VectorSubcoreMesh(core_axis_name="c", subcore_axis_name="s",
                                    num_cores=NCORES, num_subcores=NTILES),
        scratch_types=[
            pltpu.VMEM((EPT,), jnp.int32),
            pltpu.VMEM((EPT,), jnp.int32),
            pltpu.VMEM((EP_TILE,), jnp.int32),
            pltpu.VMEM((EP_TILE,), jnp.int32),
            pltpu.VMEM((EP_TILE,), jnp.int32),
            pltpu.VMEM((EP_TILE,), jnp.int32),
            pltpu.VMEM((EP_TILE,), jnp.int32),
            pltpu.VMEM((EP_TILE,), jnp.int32),
            pltpu.VMEM((NSUB, 16), jnp.int32),
        ],
        compiler_params=pltpu.CompilerParams(use_tc_tiling_on_sc=False,
                                             needs_layout_passes=False),
    )


def _mkpart(*args):
    return _make_part()(*args)


def _segsum_body(x_hbm, gidx_hbm, sidx_hbm, bounds_hbm, zeros_hbm, out_hbm,
                 gidx_v, sidx_v, bounds_v, rows_v, table_s, acc_s,
                 g0, g1, g2, g3, s0, s1, s2, s3):
    c = lax.axis_index("c")
    s = lax.axis_index("s")
    gsems = (g0, g1, g2, g3)
    ssems = (s0, s1, s2, s3)
    pltpu.sync_copy(bounds_hbm.at[s], bounds_v)

    for p in range(NPASS):
        q = c * NPASS + p
        pltpu.sync_copy(x_hbm.at[q, pl.ds(s * TRPT, TRPT)],
                        table_s.at[pl.ds(s * TRPT, TRPT)])
        for h in range(NSUB):
            pltpu.sync_copy(gidx_hbm.at[h, s], gidx_v)
            pltpu.sync_copy(sidx_hbm.at[h, s], sidx_v)
            pltpu.sync_copy(zeros_hbm.at[pl.ds(s * RPT, RPT)],
                            acc_s.at[pl.ds(s * RPT, RPT)])
            plsc.subcore_barrier()

            n = jnp.max(bounds_v[h])

            for b in range(NBUF):
                @pl.when(b < n)
                def _():
                    pltpu.async_copy(table_s.at[gidx_v.at[b]],
                                     rows_v.at[b], gsems[b])

            def outer(o, carry):
                for b in range(NBUF):
                    k = o * NBUF + b

                    @pl.when(k < n)
                    def _():
                        pltpu.make_async_copy(table_s.at[gidx_v.at[k]],
                                              rows_v.at[b], gsems[b]).wait()
                        pltpu.async_copy(rows_v.at[b],
                                         acc_s.at[sidx_v.at[k]],
                                         ssems[b], add=True)
                        bp = (b - 1) % NBUF

                        @pl.when(jnp.logical_and(k >= 1, k + NBUF - 1 < n))
                        def _():
                            pltpu.make_async_copy(
                                rows_v.at[bp], acc_s.at[sidx_v.at[k - 1]],
                                ssems[bp]).wait()
                            pltpu.async_copy(
                                table_s.at[gidx_v.at[k - 1 + NBUF]],
                                rows_v.at[bp], gsems[bp])
                return carry

            lax.fori_loop(0, (n + NBUF - 1) // NBUF, outer, 0)
            for b in range(NBUF):
                @pl.when(b < n)
                def _():
                    pltpu.make_async_copy(rows_v.at[b],
                                          acc_s.at[sidx_v.at[0]],
                                          ssems[b]).wait()

            plsc.subcore_barrier()
            @pl.when(s < 8)
            def _():
                pltpu.sync_copy(
                    acc_s.at[pl.ds(s * WRT, WRT)],
                    out_hbm.at[q, pl.ds(h * DST_H + s * WRT, WRT)])
            plsc.subcore_barrier()


@functools.cache
def _make_segsum():
    return pl.kernel(
        _segsum_body,
        out_type=jax.ShapeDtypeStruct((NQ, N_PAD, H), jnp.float32),
        mesh=plsc.VectorSubcoreMesh(core_axis_name="c", subcore_axis_name="s",
                                    num_cores=NCORES, num_subcores=NTILES),
        scratch_types=[
            pltpu.VMEM((NCHUNK, CHUNK), jnp.int32),
            pltpu.VMEM((NCHUNK, CHUNK), jnp.int32),
            pltpu.VMEM((NSUB, 16), jnp.int32),
            pltpu.VMEM((NBUF, CHUNK, H), jnp.float32),
            pltpu.VMEM_SHARED((N_PAD, H), jnp.float32),
            pltpu.VMEM_SHARED((ACC_ROWS, H), jnp.float32),
        ] + [pltpu.SemaphoreType.DMA] * (2 * NBUF),
        compiler_params=pltpu.CompilerParams(use_tc_tiling_on_sc=False,
                                             needs_layout_passes=False),
    )


def _segsum(*args):
    return _make_segsum()(*args)


def _layernorm(h, g, beta):
    mu = jnp.mean(h, axis=-1, keepdims=True)
    var = jnp.mean((h - mu) * (h - mu), axis=-1, keepdims=True)
    return (h - mu) * lax.rsqrt(var + 1e-5) * g + beta


def _cat(ref):
    return jnp.concatenate([ref[q] for q in range(NQ)], axis=-1)


def _mlp_c_body(eps_ref, x_ref, h_ref, w1_ref, b1_ref, w2_ref, b2_ref,
                g_ref, beta_ref, o_ref):
    eps1 = eps_ref[0]
    pre = eps1 * _cat(x_ref) + _cat(h_ref)
    a = jnp.dot(pre, w1_ref[0], preferred_element_type=jnp.float32) + b1_ref[0]
    a = jnp.maximum(a, 0.0)
    hb = jnp.dot(a, w2_ref[0], preferred_element_type=jnp.float32) + b2_ref[0]
    y = _layernorm(hb, g_ref[0], beta_ref[0])
    for q in range(NQ):
        o_ref[q] = y[:, q * H:(q + 1) * H]


def _mlp_l_body(eps_ref, x_ref, h_ref, w1a_ref, w1b_ref, b1_ref, w2_ref,
                b2_ref, g_ref, beta_ref, o_ref):
    eps1 = eps_ref[0]
    pre = eps1 * _cat(x_ref) + _cat(h_ref)
    up = pltpu.roll(pre, _RB - 1, 0)
    dn = pltpu.roll(pre, 1, 0)
    rid = lax.broadcasted_iota(jnp.int32, pre.shape, 0)
    sw = jnp.where((rid % 2) == 0, up, dn)
    a = (jnp.dot(pre, w1a_ref[0], preferred_element_type=jnp.float32)
         + jnp.dot(sw, w1b_ref[0], preferred_element_type=jnp.float32)
         + b1_ref[0])
    a = jnp.maximum(a, 0.0)
    hb = jnp.dot(a, w2_ref[0], preferred_element_type=jnp.float32) + b2_ref[0]
    y = _layernorm(hb, g_ref[0], beta_ref[0])
    for q in range(NQ):
        o_ref[q] = y[:, q * H:(q + 1) * H]


_RB = 1000


def _row_spec():
    return pl.BlockSpec((NQ, _RB, H), lambda i: (0, i, 0))


def _full_spec(it):
    return pl.BlockSpec((1, D, D), lambda i: (it, 0, 0))


def _vec_spec(it):
    return pl.BlockSpec((1, 1, D), lambda i: (it, 0, 0))


def _mlp_c(it, eps1, x_q, h_q, w1, b1, w2, b2, g, beta):
    return pl.pallas_call(
        _mlp_c_body,
        grid=(NC // _RB,),
        in_specs=[
            pl.BlockSpec(memory_space=pltpu.SMEM),
            _row_spec(), _row_spec(),
            _full_spec(it), _vec_spec(it), _full_spec(it), _vec_spec(it),
            _vec_spec(it), _vec_spec(it),
        ],
        out_specs=_row_spec(),
        out_shape=jax.ShapeDtypeStruct((NQ, N_PAD, H), jnp.float32),
    )(eps1, x_q, h_q, w1, b1, w2, b2, g, beta)


def _mlp_l(it, eps1, x_q, h_q, w1ab, b1, w2, b2, g, beta):
    return pl.pallas_call(
        _mlp_l_body,
        grid=(NL // _RB,),
        in_specs=[
            pl.BlockSpec(memory_space=pltpu.SMEM),
            _row_spec(), _row_spec(),
            pl.BlockSpec((1, D, D), lambda i: (2 * it, 0, 0)),
            pl.BlockSpec((1, D, D), lambda i: (2 * it + 1, 0, 0)),
            _vec_spec(it), _full_spec(it),
            _vec_spec(it), _vec_spec(it), _vec_spec(it),
        ],
        out_specs=_row_spec(),
        out_shape=jax.ShapeDtypeStruct((NQ, N_PAD, H), jnp.float32),
    )(eps1, x_q, h_q, w1ab, w1ab, b1, w2, b2, g, beta)


def kernel(edge_index, L_init, C_init, epsilon, L_W1, L_b1, L_W2, L_b2,
           L_g, L_beta, C_W1, C_b1, C_W2, C_b2, C_g, C_beta):
    f32 = jnp.float32
    i32 = jnp.int32
    src = edge_index[0].astype(i32)
    dst = edge_index[1].astype(i32)

    src2 = src.reshape(NTILES, EPT)
    dst2 = dst.reshape(NTILES, EPT)
    pg, ps, bnd = _mkpart(jnp.stack([src2, dst2]), jnp.stack([dst2, src2]))
    pg = pg.reshape(2, NSUB, NTILES, NCHUNK, CHUNK)
    ps = ps.reshape(2, NSUB, NTILES, NCHUNK, CHUNK)
    g_l2c, s_l2c, b_l2c = pg[0], ps[0], bnd[0]
    g_c2l, s_c2l, b_c2l = pg[1], ps[1], bnd[1]
    zeros_acc = jnp.zeros((ACC_ROWS, H), f32)
    eps1 = (epsilon + 1.0).astype(f32)

    def to_q(x):
        q = jnp.stack([x[:, i * H:(i + 1) * H] for i in range(NQ)])
        return jnp.concatenate(
            [q, jnp.zeros((NQ, N_PAD - q.shape[1], H), f32)], axis=1)

    scale = np.float32(1.0 / np.sqrt(D))
    lits_q = to_q(jnp.broadcast_to(L_init * scale, (NL, D)))
    cls_q = to_q(jnp.broadcast_to(C_init * scale, (NC, D)))

    L_W1r = L_W1.reshape(2 * ITERS, D, D)
    C_b1r, C_b2r = C_b1.reshape(ITERS, 1, D), C_b2.reshape(ITERS, 1, D)
    C_gr, C_betar = C_g.reshape(ITERS, 1, D), C_beta.reshape(ITERS, 1, D)
    L_b1r, L_b2r = L_b1.reshape(ITERS, 1, D), L_b2.reshape(ITERS, 1, D)
    L_gr, L_betar = L_g.reshape(ITERS, 1, D), L_beta.reshape(ITERS, 1, D)

    for i in range(ITERS):
        h_c = _segsum(lits_q, g_l2c, s_l2c, b_l2c, zeros_acc)
        cls_q = _mlp_c(i, eps1, cls_q, h_c, C_W1, C_b1r, C_W2, C_b2r,
                       C_gr, C_betar)
        h_l = _segsum(cls_q, g_c2l, s_c2l, b_c2l, zeros_acc)
        lits_q = _mlp_l(i, eps1, lits_q, h_l, L_W1r, L_b1r, L_W2, L_b2r,
                        L_gr, L_betar)

    lits_out = jnp.concatenate([lits_q[q, :NL] for q in range(NQ)], axis=-1)
    cls_out = jnp.concatenate([cls_q[q, :NC] for q in range(NQ)], axis=-1)
    return (lits_out, cls_out)


# speedup vs baseline: 1.0142x; 1.0142x over previous
import functools

import numpy as np
import jax
import jax.numpy as jnp
from jax import lax
from jax.experimental import pallas as pl
from jax.experimental.pallas import tpu as pltpu
from jax.experimental.pallas import tpu_sc as plsc

NL = 10000
NC = 10000
E = 160000
D = 256
NQ = 4
H = D // NQ
ITERS = 3

NCORES = 2
NPASS = NQ // NCORES
NTILES = 16
CHUNK = 128
EP_TILE = 10240
NCHUNK = EP_TILE // CHUNK
E_PAD = EP_TILE * NTILES
NBUF = 4

NSUB = 3
N_PAD = 10176
TRPT = N_PAD // NTILES
DST_H = N_PAD // NSUB
GARB = 64
ACC_ROWS = DST_H + GARB
RPT = ACC_ROWS // NTILES
WRT = DST_H // 8
EPT = E // NTILES
NITER = EPT // 16


def _part_body(ing_hbm, ink_hbm, pg_hbm, ps_hbm, bnd_hbm,
               gbuf, kbuf, og0, os0, og1, os1, og2, os2, bnd_v):
    d = lax.axis_index("c")
    t = lax.axis_index("s")
    pltpu.sync_copy(ing_hbm.at[d, t], gbuf)
    pltpu.sync_copy(ink_hbm.at[d, t], kbuf)

    zero16 = jnp.zeros((16,), jnp.int32)
    garb16 = DST_H + lax.iota(jnp.int32, 16)

    def fill(i, carry):
        og0[pl.ds(i * 16, 16)] = zero16
        og1[pl.ds(i * 16, 16)] = zero16
        og2[pl.ds(i * 16, 16)] = zero16
        os0[pl.ds(i * 16, 16)] = garb16
        os1[pl.ds(i * 16, 16)] = garb16
        os2[pl.ds(i * 16, 16)] = garb16
        return carry

    lax.fori_loop(0, EP_TILE // 16, fill, 0)

    def body(i, carry):
        o0, o1, o2 = carry
        kv = kbuf[pl.ds(i * 16, 16)]
        gv = gbuf[pl.ds(i * 16, 16)]
        m0 = kv < DST_H
        m2 = kv >= 2 * DST_H
        m1 = jnp.logical_not(jnp.logical_or(m0, m2))
        plsc.store_compressed(og0.at[pl.ds(o0, 16)], gv, mask=m0)
        plsc.store_compressed(os0.at[pl.ds(o0, 16)], kv, mask=m0)
        plsc.store_compressed(og1.at[pl.ds(o1, 16)], gv, mask=m1)
        plsc.store_compressed(os1.at[pl.ds(o1, 16)], kv - DST_H, mask=m1)
        plsc.store_compressed(og2.at[pl.ds(o2, 16)], gv, mask=m2)
        plsc.store_compressed(os2.at[pl.ds(o2, 16)], kv - 2 * DST_H, mask=m2)
        c0 = jnp.sum(m0.astype(jnp.int32))
        c2 = jnp.sum(m2.astype(jnp.int32))
        return (o0 + c0, o1 + (16 - c0 - c2), o2 + c2)

    cnt0, cnt1, cnt2 = lax.fori_loop(
        0, NITER, body, (jnp.int32(0), jnp.int32(0), jnp.int32(0)))
    bnd_v[0] = jnp.full((16,), (cnt0 + CHUNK - 1) // CHUNK, jnp.int32)
    bnd_v[1] = jnp.full((16,), (cnt1 + CHUNK - 1) // CHUNK, jnp.int32)
    bnd_v[2] = jnp.full((16,), (cnt2 + CHUNK - 1) // CHUNK, jnp.int32)

    pltpu.sync_copy(og0, pg_hbm.at[d, 0, t])
    pltpu.sync_copy(og1, pg_hbm.at[d, 1, t])
    pltpu.sync_copy(og2, pg_hbm.at[d, 2, t])
    pltpu.sync_copy(os0, ps_hbm.at[d, 0, t])
    pltpu.sync_copy(os1, ps_hbm.at[d, 1, t])
    pltpu.sync_copy(os2, ps_hbm.at[d, 2, t])
    pltpu.sync_copy(bnd_v, bnd_hbm.at[d, t])


@functools.cache
def _make_part():
    return pl.kernel(
        _part_body,
        out_type=(
            jax.ShapeDtypeStruct((2, NSUB, NTILES, EP_TILE), jnp.int32),
            jax.ShapeDtypeStruct((2, NSUB, NTILES, EP_TILE), jnp.int32),
            jax.ShapeDtypeStruct((2, NTILES, NSUB, 16), jnp.int32),
        ),
        mesh=plsc.VectorSubcoreMesh(core_axis_name="c", subcore_axis_name="s",
                                    num_cores=NCORES, num_subcores=NTILES),
        scratch_types=[
            pltpu.VMEM((EPT,), jnp.int32),
            pltpu.VMEM((EPT,), jnp.int32),
            pltpu.VMEM((EP_TILE,), jnp.int32),
            pltpu.VMEM((EP_TILE,), jnp.int32),
            pltpu.VMEM((EP_TILE,), jnp.int32),
            pltpu.VMEM((EP_TILE,), jnp.int32),
            pltpu.VMEM((EP_TILE,), jnp.int32),
            pltpu.VMEM((EP_TILE,), jnp.int32),
            pltpu.VMEM((NSUB, 16), jnp.int32),
        ],
        compiler_params=pltpu.CompilerParams(use_tc_tiling_on_sc=False,
                                             needs_layout_passes=False),
    )


def _mkpart(*args):
    return _make_part()(*args)


def _segsum_body(x_hbm, gidx_hbm, sidx_hbm, bounds_hbm, zeros_hbm, out_hbm,
                 gidx_v, sidx_v, bounds_v, rows_v, table_s, acc_s,
                 g0, g1, g2, g3, s0, s1, s2, s3):
    c = lax.axis_index("c")
    s = lax.axis_index("s")
    gsems = (g0, g1, g2, g3)
    ssems = (s0, s1, s2, s3)
    pltpu.sync_copy(bounds_hbm.at[s], bounds_v)

    for p in range(NPASS):
        q = c * NPASS + p
        pltpu.sync_copy(x_hbm.at[q, pl.ds(s * TRPT, TRPT)],
                        table_s.at[pl.ds(s * TRPT, TRPT)])
        for h in range(NSUB):
            pltpu.sync_copy(gidx_hbm.at[h, s], gidx_v)
            pltpu.sync_copy(sidx_hbm.at[h, s], sidx_v)
            pltpu.sync_copy(zeros_hbm.at[pl.ds(s * RPT, RPT)],
                            acc_s.at[pl.ds(s * RPT, RPT)])
            plsc.subcore_barrier()

            n = jnp.max(bounds_v[h])

            for b in range(NBUF):
                @pl.when(b < n)
                def _():
                    pltpu.async_copy(table_s.at[gidx_v.at[b]],
                                     rows_v.at[b], gsems[b])

            def outer(o, carry):
                for b in range(NBUF):
                    k = o * NBUF + b

                    @pl.when(k < n)
                    def _():
                        pltpu.make_async_copy(table_s.at[gidx_v.at[k]],
                                              rows_v.at[b], gsems[b]).wait()
                        pltpu.async_copy(rows_v.at[b],
                                         acc_s.at[sidx_v.at[k]],
                                         ssems[b], add=True)
                        bp = (b - 1) % NBUF

                        @pl.when(jnp.logical_and(k >= 1, k + NBUF - 1 < n))
                        def _():
                            pltpu.make_async_copy(
                                rows_v.at[bp], acc_s.at[sidx_v.at[k - 1]],
                                ssems[bp]).wait()
                            pltpu.async_copy(
                                table_s.at[gidx_v.at[k - 1 + NBUF]],
                                rows_v.at[bp], gsems[bp])
                return carry

            lax.fori_loop(0, (n + NBUF - 1) // NBUF, outer, 0)
            for b in range(NBUF):
                @pl.when(b < n)
                def _():
                    pltpu.make_async_copy(rows_v.at[b],
                                          acc_s.at[sidx_v.at[0]],
                                          ssems[b]).wait()

            plsc.subcore_barrier()
            @pl.when(s < 8)
            def _():
                pltpu.sync_copy(
                    acc_s.at[pl.ds(s * WRT, WRT)],
                    out_hbm.at[q, pl.ds(h * DST_H + s * WRT, WRT)])
            plsc.subcore_barrier()


@functools.cache
def _make_segsum():
    return pl.kernel(
        _segsum_body,
        out_type=jax.ShapeDtypeStruct((NQ, N_PAD, H), jnp.float32),
        mesh=plsc.VectorSubcoreMesh(core_axis_name="c", subcore_axis_name="s",
                                    num_cores=NCORES, num_subcores=NTILES),
        scratch_types=[
            pltpu.VMEM((NCHUNK, CHUNK), jnp.int32),
            pltpu.VMEM((NCHUNK, CHUNK), jnp.int32),
            pltpu.VMEM((NSUB, 16), jnp.int32),
            pltpu.VMEM((NBUF, CHUNK, H), jnp.float32),
            pltpu.VMEM_SHARED((N_PAD, H), jnp.float32),
            pltpu.VMEM_SHARED((ACC_ROWS, H), jnp.float32),
        ] + [pltpu.SemaphoreType.DMA] * (2 * NBUF),
        compiler_params=pltpu.CompilerParams(use_tc_tiling_on_sc=False,
                                             needs_layout_passes=False),
    )


def _segsum(*args):
    return _make_segsum()(*args)


def _layernorm(h, g, beta):
    mu = jnp.mean(h, axis=-1, keepdims=True)
    var = jnp.mean((h - mu) * (h - mu), axis=-1, keepdims=True)
    return (h - mu) * lax.rsqrt(var + 1e-5) * g + beta


def _cat(ref):
    return jnp.concatenate([ref[q] for q in range(NQ)], axis=-1)


def _mlp_c_body(eps_ref, x_ref, h_ref, w1_ref, b1_ref, w2_ref, b2_ref,
                g_ref, beta_ref, o_ref):
    eps1 = eps_ref[0]
    pre = eps1 * _cat(x_ref) + _cat(h_ref)
    a = jnp.dot(pre, w1_ref[...], preferred_element_type=jnp.float32) + b1_ref[...]
    a = jnp.maximum(a, 0.0)
    hb = jnp.dot(a, w2_ref[...], preferred_element_type=jnp.float32) + b2_ref[...]
    y = _layernorm(hb, g_ref[...], beta_ref[...])
    for q in range(NQ):
        o_ref[q] = y[:, q * H:(q + 1) * H]


def _mlp_l_body(eps_ref, x_ref, h_ref, w1a_ref, w1b_ref, b1_ref, w2_ref,
                b2_ref, g_ref, beta_ref, o_ref):
    eps1 = eps_ref[0]
    pre = eps1 * _cat(x_ref) + _cat(h_ref)
    up = pltpu.roll(pre, _RB - 1, 0)
    dn = pltpu.roll(pre, 1, 0)
    rid = lax.broadcasted_iota(jnp.int32, pre.shape, 0)
    sw = jnp.where((rid % 2) == 0, up, dn)
    a = (jnp.dot(pre, w1a_ref[...], preferred_element_type=jnp.float32)
         + jnp.dot(sw, w1b_ref[...], preferred_element_type=jnp.float32)
         + b1_ref[...])
    a = jnp.maximum(a, 0.0)
    hb = jnp.dot(a, w2_ref[...], preferred_element_type=jnp.float32) + b2_ref[...]
    y = _layernorm(hb, g_ref[...], beta_ref[...])
    for q in range(NQ):
        o_ref[q] = y[:, q * H:(q + 1) * H]


_RB = 1000


def _row_spec():
    return pl.BlockSpec((NQ, _RB, H), lambda i: (0, i, 0))


def _full_spec():
    return pl.BlockSpec((D, D), lambda i: (0, 0))


def _vec_spec():
    return pl.BlockSpec((1, D), lambda i: (0, 0))


def _mlp_c(eps1, x_q, h_q, w1, b1, w2, b2, g, beta):
    return pl.pallas_call(
        _mlp_c_body,
        grid=(NC // _RB,),
        in_specs=[
            pl.BlockSpec(memory_space=pltpu.SMEM),
            _row_spec(), _row_spec(),
            _full_spec(), _vec_spec(), _full_spec(), _vec_spec(),
            _vec_spec(), _vec_spec(),
        ],
        out_specs=_row_spec(),
        out_shape=jax.ShapeDtypeStruct((NQ, N_PAD, H), jnp.float32),
    )(eps1, x_q, h_q, w1, b1, w2, b2, g, beta)


def _mlp_l(eps1, x_q, h_q, w1a, w1b, b1, w2, b2, g, beta):
    return pl.pallas_call(
        _mlp_l_body,
        grid=(NL // _RB,),
        in_specs=[
            pl.BlockSpec(memory_space=pltpu.SMEM),
            _row_spec(), _row_spec(),
            _full_spec(), _full_spec(), _vec_spec(), _full_spec(),
            _vec_spec(), _vec_spec(), _vec_spec(),
        ],
        out_specs=_row_spec(),
        out_shape=jax.ShapeDtypeStruct((NQ, N_PAD, H), jnp.float32),
    )(eps1, x_q, h_q, w1a, w1b, b1, w2, b2, g, beta)


def kernel(edge_index, L_init, C_init, epsilon, L_W1, L_b1, L_W2, L_b2,
           L_g, L_beta, C_W1, C_b1, C_W2, C_b2, C_g, C_beta):
    f32 = jnp.float32
    i32 = jnp.int32
    src = edge_index[0].astype(i32)
    dst = edge_index[1].astype(i32)

    src2 = src.reshape(NTILES, EPT)
    dst2 = dst.reshape(NTILES, EPT)
    pg, ps, bnd = _mkpart(jnp.stack([src2, dst2]), jnp.stack([dst2, src2]))
    pg = pg.reshape(2, NSUB, NTILES, NCHUNK, CHUNK)
    ps = ps.reshape(2, NSUB, NTILES, NCHUNK, CHUNK)
    g_l2c, s_l2c, b_l2c = pg[0], ps[0], bnd[0]
    g_c2l, s_c2l, b_c2l = pg[1], ps[1], bnd[1]
    zeros_acc = jnp.zeros((ACC_ROWS, H), f32)
    eps1 = (epsilon + 1.0).astype(f32)

    def to_q(x):
        q = jnp.stack([x[:, i * H:(i + 1) * H] for i in range(NQ)])
        return jnp.concatenate(
            [q, jnp.zeros((NQ, N_PAD - q.shape[1], H), f32)], axis=1)

    scale = np.float32(1.0 / np.sqrt(D))
    lits_q = to_q(jnp.broadcast_to(L_init * scale, (NL, D)))
    cls_q = to_q(jnp.broadcast_to(C_init * scale, (NC, D)))

    for i in range(ITERS):
        h_c = _segsum(lits_q, g_l2c, s_l2c, b_l2c, zeros_acc)
        cls_q = _mlp_c(eps1, cls_q, h_c, C_W1[i], C_b1[i].reshape(1, D),
                       C_W2[i], C_b2[i].reshape(1, D), C_g[i].reshape(1, D),
                       C_beta[i].reshape(1, D))
        h_l = _segsum(cls_q, g_c2l, s_c2l, b_c2l, zeros_acc)
        lits_q = _mlp_l(eps1, lits_q, h_l, L_W1[i, :D], L_W1[i, D:],
                        L_b1[i].reshape(1, D), L_W2[i],
                        L_b2[i].reshape(1, D), L_g[i].reshape(1, D),
                        L_beta[i].reshape(1, D))

    lits_out = jnp.concatenate([lits_q[q, :NL] for q in range(NQ)], axis=-1)
    cls_out = jnp.concatenate([cls_q[q, :NC] for q in range(NQ)], axis=-1)
    return (lits_out, cls_out)
